# baseline (speedup 1.0000x reference)
import functools

import jax
import jax.numpy as jnp
from jax import lax
from jax.experimental import pallas as pl
from jax.experimental.pallas import tpu as pltpu
from jax.experimental.pallas import tpu_sc as plsc

D_MODEL = 1024
D_VOCAB = 100000
BATCH = 4
SEQ = 2048
N_TOK = BATCH * SEQ

NC = 2
NS = 16
NW = NC * NS
TOK_PER_W = N_TOK // NW
CHUNK = 16
NCHUNK = TOK_PER_W // CHUNK
NBUF = 7


def _sc_row_gather(W_T, x):
    mesh = plsc.VectorSubcoreMesh(core_axis_name="c", subcore_axis_name="s")

    @functools.partial(
        pl.kernel,
        out_type=jax.ShapeDtypeStruct((N_TOK, D_MODEL), jnp.float32),
        mesh=mesh,
        compiler_params=pltpu.CompilerParams(
            needs_layout_passes=False,
            use_tc_tiling_on_sc=True,
            disable_bounds_checks=True,
            disable_semaphore_checks=True,
            skip_device_barrier=True,
        ),
        scratch_types=[
            pltpu.VMEM((TOK_PER_W,), jnp.int32),
            pltpu.VMEM((NBUF, CHUNK, D_MODEL), jnp.float32),
            pltpu.SemaphoreType.DMA((NBUF,)),
            pltpu.SemaphoreType.DMA((NBUF,)),
        ],
    )
    def k(wt_hbm, x_hbm, out_hbm, idx_v, bufs, gsems, wsems):
        wid = lax.axis_index("s") * NC + lax.axis_index("c")
        base = wid * TOK_PER_W
        b = wid // (SEQ // TOK_PER_W)
        p0 = (wid % (SEQ // TOK_PER_W)) * TOK_PER_W
        pltpu.sync_copy(x_hbm.at[b, pl.ds(p0, TOK_PER_W)], idx_v)

        def start_gather(ch):
            off = pl.multiple_of(ch * CHUNK, CHUNK)
            pltpu.async_copy(
                wt_hbm.at[idx_v.at[pl.ds(off, CHUNK)]],
                bufs.at[ch % NBUF],
                gsems.at[ch % NBUF],
            )

        def wait_gather(ch):
            pltpu.make_async_copy(
                wt_hbm.at[idx_v.at[pl.ds(0, CHUNK)]],
                bufs.at[ch % NBUF],
                gsems.at[ch % NBUF],
            ).wait()

        def start_scatter(ch):
            off = pl.multiple_of(base + ch * CHUNK, CHUNK)
            pltpu.async_copy(
                bufs.at[ch % NBUF],
                out_hbm.at[pl.ds(off, CHUNK)],
                wsems.at[ch % NBUF],
            )

        def wait_scatter(ch):
            pltpu.make_async_copy(
                bufs.at[ch % NBUF],
                out_hbm.at[pl.ds(base, CHUNK)],
                wsems.at[ch % NBUF],
            ).wait()

        def prime(ch, carry):
            start_gather(ch)
            return carry

        lax.fori_loop(0, NBUF - 1, prime, 0)

        def body(ch, carry):
            nxt = ch + NBUF - 1

            @pl.when(nxt < NCHUNK)
            def _():
                @pl.when(ch >= 1)
                def _():
                    wait_scatter(ch - 1)

                start_gather(nxt)

            wait_gather(ch)
            start_scatter(ch)
            return carry

        lax.fori_loop(0, NCHUNK, body, 0)

        def drain(ch, carry):
            wait_scatter(ch)
            return carry

        lax.fori_loop(NCHUNK - NBUF, NCHUNK, drain, 0)

    return k(W_T, x)


@jax.jit
def kernel(x, W_E):
    out = _sc_row_gather(W_E.T, x.astype(jnp.int32))
    return out.reshape(BATCH, SEQ, D_MODEL)


# speedup vs baseline: 1.0028x; 1.0028x over previous
import functools

import jax
import jax.numpy as jnp
from jax import lax
from jax.experimental import pallas as pl
from jax.experimental.pallas import tpu as pltpu
from jax.experimental.pallas import tpu_sc as plsc

D_MODEL = 1024
D_VOCAB = 100000
BATCH = 4
SEQ = 2048
N_TOK = BATCH * SEQ

NC = 2
NS = 16
NW = NC * NS
TOK_PER_W = N_TOK // NW
CHUNK = 16
NCHUNK = TOK_PER_W // CHUNK
NBUF = 7


def _sc_row_gather(W_T, x):
    mesh = plsc.VectorSubcoreMesh(core_axis_name="c", subcore_axis_name="s")

    @functools.partial(
        pl.kernel,
        out_type=jax.ShapeDtypeStruct((N_TOK, D_MODEL), jnp.float32),
        mesh=mesh,
        compiler_params=pltpu.CompilerParams(
            needs_layout_passes=False, use_tc_tiling_on_sc=True
        ),
        scratch_types=[
            pltpu.VMEM((TOK_PER_W,), jnp.int32),
            pltpu.VMEM((NBUF, CHUNK, D_MODEL), jnp.float32),
            pltpu.SemaphoreType.DMA((NBUF,)),
            pltpu.SemaphoreType.DMA((NBUF,)),
        ],
    )
    def k(wt_hbm, x_hbm, out_hbm, idx_v, bufs, gsems, wsems):
        wid = lax.axis_index("s") * NC + lax.axis_index("c")
        base = wid * TOK_PER_W
        b = wid // (SEQ // TOK_PER_W)
        p0 = (wid % (SEQ // TOK_PER_W)) * TOK_PER_W
        pltpu.sync_copy(x_hbm.at[b, pl.ds(p0, TOK_PER_W)], idx_v)

        def start_gather(ch):
            off = pl.multiple_of(ch * CHUNK, CHUNK)
            pltpu.async_copy(
                wt_hbm.at[idx_v.at[pl.ds(off, CHUNK)]],
                bufs.at[ch % NBUF],
                gsems.at[ch % NBUF],
            )

        def wait_gather(ch):
            pltpu.make_async_copy(
                wt_hbm.at[idx_v.at[pl.ds(0, CHUNK)]],
                bufs.at[ch % NBUF],
                gsems.at[ch % NBUF],
            ).wait()

        def start_scatter(ch):
            off = pl.multiple_of(base + ch * CHUNK, CHUNK)
            pltpu.async_copy(
                bufs.at[ch % NBUF],
                out_hbm.at[pl.ds(off, CHUNK)],
                wsems.at[ch % NBUF],
            )

        def wait_scatter(ch):
            pltpu.make_async_copy(
                bufs.at[ch % NBUF],
                out_hbm.at[pl.ds(base, CHUNK)],
                wsems.at[ch % NBUF],
            ).wait()

        def prime(ch, carry):
            start_gather(ch)
            return carry

        lax.fori_loop(0, NBUF - 1, prime, 0)

        def body(ch, carry):
            nxt = ch + NBUF - 1

            @pl.when(nxt < NCHUNK)
            def _():
                @pl.when(ch >= 1)
                def _():
                    wait_scatter(ch - 1)

                start_gather(nxt)

            wait_gather(ch)
            start_scatter(ch)
            return carry

        lax.fori_loop(0, NCHUNK, body, 0)

        def drain(ch, carry):
            wait_scatter(ch)
            return carry

        lax.fori_loop(NCHUNK - NBUF, NCHUNK, drain, 0)

    return k(W_T, x)


@jax.jit
def kernel(x, W_E):
    out = _sc_row_gather(W_E.T, x.astype(jnp.int32))
    return out.reshape(BATCH, SEQ, D_MODEL)
